# baseline (speedup 1.0000x reference)
import jax
import jax.numpy as jnp
from jax import lax
from jax.experimental import pallas as pl
from jax.experimental.pallas import tpu as pltpu

T = 2048
D = 1024
E = 8
B = 128
NB = T // B
NBUF = 3
CK = 8


def _gates_kernel(x_ref, w_ref, gates_ref, ent_ref):
    x = x_ref[...]
    w = w_ref[...]
    logits = lax.dot_general(x, w, (((1,), (1,)), ((), ())),
                             preferred_element_type=jnp.float32)
    m = jnp.max(logits, axis=1, keepdims=True)
    ex = jnp.exp(logits - m)
    g = ex / jnp.sum(ex, axis=1, keepdims=True)
    gates_ref[...] = g
    ent = -jnp.sum(g * jnp.log(g + 1e-9), axis=1)
    ent_ref[0, 0] = jnp.sum(ent) / jnp.float32(T)


def _diag_kernel(gates_ref, mask_ref):
    i = pl.program_id(0)
    g = gates_ref[pl.ds(i * B, B), :]

    mask_ref[...] = jnp.zeros((B, E * T), jnp.bool_)
    row2 = lax.broadcasted_iota(jnp.int32, (B, B), 0)
    col2 = lax.broadcasted_iota(jnp.int32, (B, B), 1)
    d2 = row2 == col2
    for e in range(E):
        ge = g[:, e][:, None]
        mask_ref[:, pl.ds(e * T + i * B, B)] = jnp.logical_and(d2, ge != 0.0)


def kernel(input, W):
    gates, ent = pl.pallas_call(
        _gates_kernel,
        out_shape=(
            jax.ShapeDtypeStruct((T, E), jnp.float32),
            jax.ShapeDtypeStruct((1, 1), jnp.float32),
        ),
        out_specs=(
            pl.BlockSpec(memory_space=pltpu.VMEM),
            pl.BlockSpec(memory_space=pltpu.SMEM),
        ),
    )(input, W)

    mask2 = pl.pallas_call(
        _diag_kernel,
        grid=(NB,),
        in_specs=(pl.BlockSpec(memory_space=pltpu.VMEM),),
        out_specs=pl.BlockSpec((B, E * T), lambda i: (i, 0)),
        out_shape=jax.ShapeDtypeStruct((T, E * T), jnp.bool_),
    )(gates)
    comb = jnp.zeros((T, E, T), jnp.float32)

    mask = mask2.reshape(T, E, T)
    l_aux = jnp.zeros((1,), dtype=jnp.float32)
    return (l_aux, comb, mask, ent[0, 0])


# speedup vs baseline: 1.3824x; 1.3824x over previous
import jax
import jax.numpy as jnp
from jax import lax
from jax.experimental import pallas as pl
from jax.experimental.pallas import tpu as pltpu

T = 2048
D = 1024
E = 8
B = 128
NB = T // B
NBUF = 2
CK = 8


def _gates_kernel(x_ref, w_ref, gates_ref, ent_ref):
    x = x_ref[...]
    w = w_ref[...]
    logits = lax.dot_general(x, w, (((1,), (1,)), ((), ())),
                             preferred_element_type=jnp.float32)
    m = jnp.max(logits, axis=1, keepdims=True)
    ex = jnp.exp(logits - m)
    g = ex / jnp.sum(ex, axis=1, keepdims=True)
    gates_ref[...] = g
    ent = -jnp.sum(g * jnp.log(g + 1e-9), axis=1)
    ent_ref[0, 0] = jnp.sum(ent) / jnp.float32(T)


def _diag_kernel(gates_ref, comb_ref, mask_ref, cbuf, csem):
    i = pl.program_id(0)
    b = lax.rem(i, NBUF)

    def sub_copies(buf_idx, step):
        bk = B // CK
        return [
            pltpu.make_async_copy(
                cbuf.at[buf_idx, pl.ds(k * bk, bk)],
                comb_ref.at[pl.ds(step * B + k * bk, bk)],
                csem.at[buf_idx],
            )
            for k in range(CK)
        ]

    @pl.when(i >= NBUF)
    def _reclaim():
        for c in sub_copies(b, i - NBUF):
            c.wait()
        cbuf[b, :, :, pl.ds((i - NBUF) * B, B)] = jnp.zeros(
            (B, E, B), jnp.float32)

    @pl.when(i < NBUF)
    def _init():
        cbuf[b] = jnp.zeros((B, E, T), jnp.float32)

    g = gates_ref[pl.ds(i * B, B), :]
    row = lax.broadcasted_iota(jnp.int32, (B, E, B), 0)
    col = lax.broadcasted_iota(jnp.int32, (B, E, B), 2)
    d = row == col
    gb = g[:, :, None]
    cbuf[b, :, :, pl.ds(i * B, B)] = jnp.where(d, gb, 0.0)
    for c in sub_copies(b, i):
        c.start()

    mask_ref[pl.ds(i * B, B), :] = jnp.zeros((B, E * T), jnp.int8)
    row2 = lax.broadcasted_iota(jnp.int32, (B, B), 0)
    col2 = lax.broadcasted_iota(jnp.int32, (B, B), 1)
    d2 = row2 == col2
    for e in range(E):
        ge = g[:, e][:, None]
        mask_ref[pl.ds(i * B, B), pl.ds(e * T + i * B, B)] = jnp.logical_and(
            d2, ge != 0.0).astype(jnp.int8)

    @pl.when(i == NB - 1)
    def _drain():
        for s in range(NB - NBUF, NB):
            for c in sub_copies(s % NBUF, s):
                c.wait()


def kernel(input, W):
    gates, ent = pl.pallas_call(
        _gates_kernel,
        out_shape=(
            jax.ShapeDtypeStruct((T, E), jnp.float32),
            jax.ShapeDtypeStruct((1, 1), jnp.float32),
        ),
        out_specs=(
            pl.BlockSpec(memory_space=pltpu.VMEM),
            pl.BlockSpec(memory_space=pltpu.SMEM),
        ),
    )(input, W)

    comb, mask_i8 = pl.pallas_call(
        _diag_kernel,
        grid=(NB,),
        in_specs=(pl.BlockSpec(memory_space=pltpu.VMEM),),
        out_specs=(
            pl.BlockSpec(memory_space=pl.ANY),
            pl.BlockSpec(memory_space=pltpu.VMEM),
        ),
        out_shape=(
            jax.ShapeDtypeStruct((T, E, T), jnp.float32),
            jax.ShapeDtypeStruct((T, E * T), jnp.int8),
        ),
        scratch_shapes=[
            pltpu.VMEM((NBUF, B, E, T), jnp.float32),
            pltpu.SemaphoreType.DMA((NBUF,)),
        ],
    )(gates)

    mask = mask_i8.astype(jnp.bool_).reshape(T, E, T)
    l_aux = jnp.zeros((1,), dtype=jnp.float32)
    return (l_aux, comb, mask, ent[0, 0])


# speedup vs baseline: 1.5548x; 1.1248x over previous
import jax
import jax.numpy as jnp
from jax import lax
from jax.experimental import pallas as pl
from jax.experimental.pallas import tpu as pltpu

T = 2048
D = 1024
E = 8
B = 128
NB = T // B
NBUF = 3
CK = 8
MK = 2


def _gates_mask_kernel(x_ref, w_ref, gates_ref, ent_ref, mask_ref,
                       gsc, mbuf, msem):
    i = pl.program_id(0)
    b = lax.rem(i, NBUF)

    @pl.when(i == 0)
    def _compute_gates():
        x = x_ref[...]
        w = w_ref[...]
        logits = lax.dot_general(x, w, (((1,), (1,)), ((), ())),
                                 preferred_element_type=jnp.float32)
        m = jnp.max(logits, axis=1, keepdims=True)
        ex = jnp.exp(logits - m)
        g = ex / jnp.sum(ex, axis=1, keepdims=True)
        gates_ref[...] = g
        gsc[...] = g
        ent = -jnp.sum(g * jnp.log(g + 1e-9), axis=1)
        ent_ref[0, 0] = jnp.sum(ent) / jnp.float32(T)

    def sub_copies(buf_idx, step):
        bk = B // MK
        return [
            pltpu.make_async_copy(
                mbuf.at[buf_idx, pl.ds(k * bk, bk)],
                mask_ref.at[pl.ds(step * B + k * bk, bk)],
                msem.at[buf_idx],
            )
            for k in range(MK)
        ]

    @pl.when(i >= NBUF)
    def _reclaim():
        for c in sub_copies(b, i - NBUF):
            c.wait()

    g = gsc[pl.ds(i * B, B), :]
    mbuf[b] = jnp.zeros((B, E * T), jnp.int8)
    row2 = lax.broadcasted_iota(jnp.int32, (B, B), 0)
    col2 = lax.broadcasted_iota(jnp.int32, (B, B), 1)
    d2 = row2 == col2
    for e in range(E):
        ge = g[:, e][:, None]
        mbuf[b, :, pl.ds(e * T + i * B, B)] = jnp.logical_and(
            d2, ge != 0.0).astype(jnp.int8)
    for c in sub_copies(b, i):
        c.start()

    @pl.when(i == NB - 1)
    def _drain():
        for s in range(NB - NBUF, NB):
            for c in sub_copies(s % NBUF, s):
                c.wait()


def _diag_kernel(gates_ref, comb_ref, cbuf, csem):
    i = pl.program_id(0)
    b = lax.rem(i, NBUF)

    def sub_copies(buf_idx, step):
        bk = B // CK
        return [
            pltpu.make_async_copy(
                cbuf.at[buf_idx, pl.ds(k * bk, bk)],
                comb_ref.at[pl.ds(step * B + k * bk, bk)],
                csem.at[buf_idx],
            )
            for k in range(CK)
        ]

    @pl.when(i >= NBUF)
    def _reclaim():
        for c in sub_copies(b, i - NBUF):
            c.wait()
        cbuf[b, :, :, pl.ds((i - NBUF) * B, B)] = jnp.zeros(
            (B, E, B), jnp.float32)

    @pl.when(i < NBUF)
    def _init():
        cbuf[b] = jnp.zeros((B, E, T), jnp.float32)

    g = gates_ref[pl.ds(i * B, B), :]
    row = lax.broadcasted_iota(jnp.int32, (B, E, B), 0)
    col = lax.broadcasted_iota(jnp.int32, (B, E, B), 2)
    d = row == col
    gb = g[:, :, None]
    cbuf[b, :, :, pl.ds(i * B, B)] = jnp.where(d, gb, 0.0)
    for c in sub_copies(b, i):
        c.start()

    @pl.when(i == NB - 1)
    def _drain():
        for s in range(NB - NBUF, NB):
            for c in sub_copies(s % NBUF, s):
                c.wait()


def kernel(input, W):
    gates, ent, mask_i8 = pl.pallas_call(
        _gates_mask_kernel,
        grid=(NB,),
        in_specs=(
            pl.BlockSpec(memory_space=pltpu.VMEM),
            pl.BlockSpec(memory_space=pltpu.VMEM),
        ),
        out_specs=(
            pl.BlockSpec(memory_space=pltpu.VMEM),
            pl.BlockSpec(memory_space=pltpu.SMEM),
            pl.BlockSpec(memory_space=pl.ANY),
        ),
        out_shape=(
            jax.ShapeDtypeStruct((T, E), jnp.float32),
            jax.ShapeDtypeStruct((1, 1), jnp.float32),
            jax.ShapeDtypeStruct((T, E * T), jnp.int8),
        ),
        scratch_shapes=[
            pltpu.VMEM((T, E), jnp.float32),
            pltpu.VMEM((NBUF, B, E * T), jnp.int8),
            pltpu.SemaphoreType.DMA((NBUF,)),
        ],
    )(input, W)

    mask = mask_i8.astype(jnp.bool_).reshape(T, E, T)

    comb = pl.pallas_call(
        _diag_kernel,
        grid=(NB,),
        in_specs=(pl.BlockSpec(memory_space=pltpu.VMEM),),
        out_specs=pl.BlockSpec(memory_space=pl.ANY),
        out_shape=jax.ShapeDtypeStruct((T, E, T), jnp.float32),
        scratch_shapes=[
            pltpu.VMEM((NBUF, B, E, T), jnp.float32),
            pltpu.SemaphoreType.DMA((NBUF,)),
        ],
    )(gates)

    l_aux = jnp.zeros((1,), dtype=jnp.float32)
    return (l_aux, comb, mask, ent[0, 0])


# speedup vs baseline: 2.2064x; 1.4191x over previous
import jax
import jax.numpy as jnp
from jax import lax
from jax.experimental import pallas as pl
from jax.experimental.pallas import tpu as pltpu

T = 2048
D = 1024
E = 8
B = 128
NB = T // B
NBUF = 3
CK = 8
MK = 2


def _gates_mask_kernel(x_ref, w_ref, gates_ref, ent_ref, mask_ref,
                       gsc, mbuf, msem):
    i = pl.program_id(0)
    b = lax.rem(i, NBUF)

    @pl.when(i == 0)
    def _compute_gates():
        x = x_ref[...]
        w = w_ref[...]
        logits = lax.dot_general(x, w, (((1,), (1,)), ((), ())),
                                 preferred_element_type=jnp.float32)
        m = jnp.max(logits, axis=1, keepdims=True)
        ex = jnp.exp(logits - m)
        g = ex / jnp.sum(ex, axis=1, keepdims=True)
        gates_ref[...] = g
        gsc[...] = g
        ent = -jnp.sum(g * jnp.log(g + 1e-9), axis=1)
        ent_ref[0, 0] = jnp.sum(ent) / jnp.float32(T)

    mask_flat = mask_ref.reshape(T * E, T)

    def sub_copies(buf_idx, step):
        bk = B * E // MK
        return [
            pltpu.make_async_copy(
                mbuf.at[buf_idx, pl.ds(k * bk, bk)],
                mask_flat.at[pl.ds((step * B * E) + k * bk, bk)],
                msem.at[buf_idx],
            )
            for k in range(MK)
        ]

    @pl.when(i >= NBUF)
    def _reclaim():
        for c in sub_copies(b, i - NBUF):
            c.wait()

    g = gsc[pl.ds(i * B, B), :]
    gnzT = (g.T != 0.0).astype(jnp.int8)
    gtile = jnp.broadcast_to(gnzT[None, :, :], (B, E, B)).reshape(B * E, B)
    mbuf[b] = jnp.zeros((B * E, T), jnp.int8)
    row2 = lax.broadcasted_iota(jnp.int32, (B * E, B), 0)
    col2 = lax.broadcasted_iota(jnp.int32, (B * E, B), 1)
    d2 = lax.div(row2, E) == col2
    mbuf[b, :, pl.ds(i * B, B)] = jnp.where(d2, gtile, 0)
    for c in sub_copies(b, i):
        c.start()

    @pl.when(i == NB - 1)
    def _drain():
        for s in range(NB - NBUF, NB):
            for c in sub_copies(s % NBUF, s):
                c.wait()


def _diag_kernel(gates_ref, comb_ref, cbuf, csem):
    i = pl.program_id(0)
    b = lax.rem(i, NBUF)

    def sub_copies(buf_idx, step):
        bk = B // CK
        return [
            pltpu.make_async_copy(
                cbuf.at[buf_idx, pl.ds(k * bk, bk)],
                comb_ref.at[pl.ds(step * B + k * bk, bk)],
                csem.at[buf_idx],
            )
            for k in range(CK)
        ]

    @pl.when(i >= NBUF)
    def _reclaim():
        for c in sub_copies(b, i - NBUF):
            c.wait()
        cbuf[b, :, :, pl.ds((i - NBUF) * B, B)] = jnp.zeros(
            (B, E, B), jnp.float32)

    @pl.when(i < NBUF)
    def _init():
        cbuf[b] = jnp.zeros((B, E, T), jnp.float32)

    g = gates_ref[pl.ds(i * B, B), :]
    row = lax.broadcasted_iota(jnp.int32, (B, E, B), 0)
    col = lax.broadcasted_iota(jnp.int32, (B, E, B), 2)
    d = row == col
    gb = g[:, :, None]
    cbuf[b, :, :, pl.ds(i * B, B)] = jnp.where(d, gb, 0.0)
    for c in sub_copies(b, i):
        c.start()

    @pl.when(i == NB - 1)
    def _drain():
        for s in range(NB - NBUF, NB):
            for c in sub_copies(s % NBUF, s):
                c.wait()


def kernel(input, W):
    gates, ent, mask_i8 = pl.pallas_call(
        _gates_mask_kernel,
        grid=(NB,),
        in_specs=(
            pl.BlockSpec(memory_space=pltpu.VMEM),
            pl.BlockSpec(memory_space=pltpu.VMEM),
        ),
        out_specs=(
            pl.BlockSpec(memory_space=pltpu.VMEM),
            pl.BlockSpec(memory_space=pltpu.SMEM),
            pl.BlockSpec(memory_space=pl.ANY),
        ),
        out_shape=(
            jax.ShapeDtypeStruct((T, E), jnp.float32),
            jax.ShapeDtypeStruct((1, 1), jnp.float32),
            jax.ShapeDtypeStruct((T, E, T), jnp.int8),
        ),
        scratch_shapes=[
            pltpu.VMEM((T, E), jnp.float32),
            pltpu.VMEM((NBUF, B * E, T), jnp.int8),
            pltpu.SemaphoreType.DMA((NBUF,)),
        ],
    )(input, W)

    mask = mask_i8.astype(jnp.bool_)

    comb = pl.pallas_call(
        _diag_kernel,
        grid=(NB,),
        in_specs=(pl.BlockSpec(memory_space=pltpu.VMEM),),
        out_specs=pl.BlockSpec(memory_space=pl.ANY),
        out_shape=jax.ShapeDtypeStruct((T, E, T), jnp.float32),
        scratch_shapes=[
            pltpu.VMEM((NBUF, B, E, T), jnp.float32),
            pltpu.SemaphoreType.DMA((NBUF,)),
        ],
    )(gates)

    l_aux = jnp.zeros((1,), dtype=jnp.float32)
    return (l_aux, comb, mask, ent[0, 0])


# speedup vs baseline: 2.2684x; 1.0281x over previous
import jax
import jax.numpy as jnp
from jax import lax
from jax.experimental import pallas as pl
from jax.experimental.pallas import tpu as pltpu

T = 2048
D = 1024
E = 8
B = 128
NB = T // B
NBUF = 3
CK = 8
MK = 2


def _kgate_kernel(x_ref, w_ref, gates_ref, ent_ref, comb_ref, mask_ref,
                  gsc, cbuf, mbuf, csem, msem):
    i = pl.program_id(0)
    b = lax.rem(i, NBUF)
    mask_flat = mask_ref.reshape(T * E, T)

    @pl.when(i == 0)
    def _compute_gates():
        x = x_ref[...]
        w = w_ref[...]
        logits = lax.dot_general(x, w, (((1,), (1,)), ((), ())),
                                 preferred_element_type=jnp.float32)
        m = jnp.max(logits, axis=1, keepdims=True)
        ex = jnp.exp(logits - m)
        g = ex / jnp.sum(ex, axis=1, keepdims=True)
        gates_ref[...] = g
        gsc[...] = g
        ent = -jnp.sum(g * jnp.log(g + 1e-9), axis=1)
        ent_ref[0, 0] = jnp.sum(ent) / jnp.float32(T)

    def sub_copies(buf_idx, step):
        copies = []
        bk = B // CK
        for k in range(CK):
            copies.append(pltpu.make_async_copy(
                cbuf.at[buf_idx, pl.ds(k * bk, bk)],
                comb_ref.at[pl.ds(step * B + k * bk, bk)],
                csem.at[buf_idx],
            ))
        bk = B * E // MK
        for k in range(MK):
            copies.append(pltpu.make_async_copy(
                mbuf.at[buf_idx, pl.ds(k * bk, bk)],
                mask_flat.at[pl.ds(step * B * E + k * bk, bk)],
                msem.at[buf_idx],
            ))
        return copies

    @pl.when(i >= NBUF)
    def _reclaim():
        for c in sub_copies(b, i - NBUF):
            c.wait()
        cbuf[b, :, :, pl.ds((i - NBUF) * B, B)] = jnp.zeros(
            (B, E, B), jnp.float32)
        mbuf[b, :, pl.ds((i - NBUF) * B, B)] = jnp.zeros(
            (B * E, B), jnp.int8)

    @pl.when(i < NBUF)
    def _init():
        cbuf[b] = jnp.zeros((B, E, T), jnp.float32)
        mbuf[b] = jnp.zeros((B * E, T), jnp.int8)

    g = gsc[pl.ds(i * B, B), :]

    row = lax.broadcasted_iota(jnp.int32, (B, E, B), 0)
    col = lax.broadcasted_iota(jnp.int32, (B, E, B), 2)
    d = row == col
    gb = g[:, :, None]
    cbuf[b, :, :, pl.ds(i * B, B)] = jnp.where(d, gb, 0.0)

    gnzT = (g.T != 0.0).astype(jnp.int8)
    gtile = jnp.broadcast_to(gnzT[None, :, :], (B, E, B)).reshape(B * E, B)
    row2 = lax.broadcasted_iota(jnp.int32, (B * E, B), 0)
    col2 = lax.broadcasted_iota(jnp.int32, (B * E, B), 1)
    d2 = lax.div(row2, E) == col2
    mbuf[b, :, pl.ds(i * B, B)] = jnp.where(d2, gtile, 0)

    for c in sub_copies(b, i):
        c.start()

    @pl.when(i == NB - 1)
    def _drain():
        for s in range(NB - NBUF, NB):
            for c in sub_copies(s % NBUF, s):
                c.wait()


def kernel(input, W):
    gates, ent, comb, mask_i8 = pl.pallas_call(
        _kgate_kernel,
        grid=(NB,),
        in_specs=(
            pl.BlockSpec(memory_space=pltpu.VMEM),
            pl.BlockSpec(memory_space=pltpu.VMEM),
        ),
        out_specs=(
            pl.BlockSpec(memory_space=pltpu.VMEM),
            pl.BlockSpec(memory_space=pltpu.SMEM),
            pl.BlockSpec(memory_space=pl.ANY),
            pl.BlockSpec(memory_space=pl.ANY),
        ),
        out_shape=(
            jax.ShapeDtypeStruct((T, E), jnp.float32),
            jax.ShapeDtypeStruct((1, 1), jnp.float32),
            jax.ShapeDtypeStruct((T, E, T), jnp.float32),
            jax.ShapeDtypeStruct((T, E, T), jnp.int8),
        ),
        scratch_shapes=[
            pltpu.VMEM((T, E), jnp.float32),
            pltpu.VMEM((NBUF, B, E, T), jnp.float32),
            pltpu.VMEM((NBUF, B * E, T), jnp.int8),
            pltpu.SemaphoreType.DMA((NBUF,)),
            pltpu.SemaphoreType.DMA((NBUF,)),
        ],
    )(input, W)

    mask = mask_i8.astype(jnp.bool_)
    l_aux = jnp.zeros((1,), dtype=jnp.float32)
    return (l_aux, comb, mask, ent[0, 0])
